# Initial kernel scaffold; baseline (speedup 1.0000x reference)
import jax
import jax.numpy as jnp
from jax.experimental import pallas as pl


def kernel(x, edge_index, batch, W1, b1, W2, b2, Wg, bg, Wf, bf):
    raise NotImplementedError("write your pallas kernel here")


# speedup vs baseline: 9.8433x; 9.8433x over previous
import functools

import jax
import jax.numpy as jnp
from jax import lax
from jax.experimental import pallas as pl
from jax.experimental.pallas import tpu as pltpu
from jax.experimental.pallas import tpu_sc as plsc

N = 10000
N_PAD = 10240
E = 320000
F = 128
FH = F // 2
G = 64
NC = 2
NS = 16
E_PER_TILE = E // NS
CHUNK = 80
N_CHUNKS = E_PER_TILE // CHUNK
ROWS_PER_TILE = N_PAD // NS
HW = 16

_mesh = plsc.VectorSubcoreMesh(core_axis_name="c", subcore_axis_name="s")


@functools.partial(
    pl.kernel,
    mesh=_mesh,
    compiler_params=pltpu.CompilerParams(use_tc_tiling_on_sc=False),
    out_type=jax.ShapeDtypeStruct((NC, N_PAD, HW), jnp.float32),
    scratch_types=[
        pltpu.VMEM((CHUNK,), jnp.int32),
        pltpu.VMEM((CHUNK, HW), jnp.float32),
        pltpu.VMEM((ROWS_PER_TILE, HW), jnp.float32),
        pltpu.VMEM_SHARED((N_PAD, HW), jnp.float32),
    ],
)
def _deg_kernel(col_hbm, out_hbm, cidx, ones_v, stage_v, hist_sh):
    c = lax.axis_index("c")
    s = lax.axis_index("s")

    one16 = jnp.ones((HW,), jnp.float32)
    zero16 = jnp.zeros((HW,), jnp.float32)

    def init_rows(i, _):
        ones_v[i, :] = one16
        return 0

    lax.fori_loop(0, CHUNK, init_rows, 0)

    def zero_rows(i, _):
        stage_v[i, :] = zero16
        return 0

    lax.fori_loop(0, ROWS_PER_TILE, zero_rows, 0)

    my_rows = pl.multiple_of(s * ROWS_PER_TILE, 8)
    pltpu.sync_copy(stage_v, hist_sh.at[pl.ds(my_rows, ROWS_PER_TILE)])
    plsc.subcore_barrier()

    tile_base = (c * NS + s) * (E_PER_TILE // 2)

    def body(j, _):
        base = pl.multiple_of(tile_base + j * CHUNK, 8)
        pltpu.sync_copy(col_hbm.at[pl.ds(base, CHUNK)], cidx)
        pltpu.sync_copy(ones_v, hist_sh.at[cidx], add=True)
        return 0

    lax.fori_loop(0, N_CHUNKS // 2, body, 0)
    plsc.subcore_barrier()

    pltpu.sync_copy(hist_sh.at[pl.ds(my_rows, ROWS_PER_TILE)], stage_v)
    pltpu.sync_copy(stage_v, out_hbm.at[c, pl.ds(my_rows, ROWS_PER_TILE)])


@functools.partial(
    pl.kernel,
    mesh=_mesh,
    compiler_params=pltpu.CompilerParams(use_tc_tiling_on_sc=False),
    out_type=jax.ShapeDtypeStruct((NC, N_PAD, FH), jnp.float32),
    scratch_types=[
        pltpu.VMEM((CHUNK,), jnp.int32),
        pltpu.VMEM((CHUNK,), jnp.int32),
        pltpu.VMEM((CHUNK, FH), jnp.float32),
        pltpu.VMEM((ROWS_PER_TILE, FH), jnp.float32),
        pltpu.VMEM_SHARED((N_PAD, FH), jnp.float32),
        pltpu.SemaphoreType.DMA,
    ],
)
def _scatter_kernel(u2_hbm, row_hbm, col_hbm, out_hbm, ridx, cidx, rows_v,
                    stage_v, acc_sh, sem):
    c = lax.axis_index("c")
    s = lax.axis_index("s")

    zero16 = jnp.zeros((16,), jnp.float32)

    def zero_rows(i, _):
        for j in range(FH // 16):
            stage_v[i, pl.ds(j * 16, 16)] = zero16
        return 0

    lax.fori_loop(0, ROWS_PER_TILE, zero_rows, 0)

    my_rows = pl.multiple_of(s * ROWS_PER_TILE, 8)
    pltpu.sync_copy(stage_v, acc_sh.at[pl.ds(my_rows, ROWS_PER_TILE)])
    plsc.subcore_barrier()

    tile_base = s * E_PER_TILE

    def body(j, _):
        base = pl.multiple_of(tile_base + j * CHUNK, 8)
        pltpu.sync_copy(row_hbm.at[pl.ds(base, CHUNK)], ridx)
        pltpu.sync_copy(col_hbm.at[pl.ds(base, CHUNK)], cidx)
        for k in range(CHUNK // 16):
            sl = pl.ds(k * 16, 16)
            ridx[sl] = ridx[sl] * 2 + c
        pltpu.async_copy(u2_hbm.at[ridx], rows_v, sem).wait()
        pltpu.sync_copy(rows_v, acc_sh.at[cidx], add=True)
        return 0

    lax.fori_loop(0, N_CHUNKS, body, 0)
    plsc.subcore_barrier()

    pltpu.sync_copy(acc_sh.at[pl.ds(my_rows, ROWS_PER_TILE)], stage_v)
    pltpu.sync_copy(stage_v, out_hbm.at[c, pl.ds(my_rows, ROWS_PER_TILE)])


RB = 2000
NB = N // RB
_P = jax.lax.Precision.HIGHEST


def _dot(a, b):
    return lax.dot_general(a, b, (((1,), (0,)), ((), ())), precision=_P,
                           preferred_element_type=jnp.float32)


def _prep_body(hist_ref, x_ref, u1_ref, dinv_ref):
    h = hist_ref[...]
    deg = 1.0 + h[0, :, 0:1] + h[1, :, 0:1]
    dinv = lax.rsqrt(deg)
    dinv_ref[...] = dinv
    u1_ref[...] = x_ref[...] * dinv


def _prep_call(hist, x):
    return pl.pallas_call(
        _prep_body,
        grid=(NB,),
        in_specs=[
            pl.BlockSpec((NC, RB, HW), lambda i: (0, i, 0)),
            pl.BlockSpec((RB, F), lambda i: (i, 0)),
        ],
        out_specs=[
            pl.BlockSpec((RB, F), lambda i: (i, 0)),
            pl.BlockSpec((RB, 1), lambda i: (i, 0)),
        ],
        out_shape=[
            jax.ShapeDtypeStruct((N, F), jnp.float32),
            jax.ShapeDtypeStruct((N, 1), jnp.float32),
        ],
    )(hist, x)


def _mid_body(s_ref, u1_ref, dinv_ref, W1_ref, b1_ref, u2_ref):
    sp = s_ref[...]
    s_full = jnp.concatenate([sp[0], sp[1]], axis=-1)
    dinv = dinv_ref[...]
    t1 = dinv * (s_full + u1_ref[...])
    h = jnp.maximum(_dot(t1, W1_ref[...]) + b1_ref[...], 0.0)
    u2_ref[...] = dinv * h


def _mid_call(s1, u1, dinv, W1, b1):
    return pl.pallas_call(
        _mid_body,
        grid=(NB,),
        in_specs=[
            pl.BlockSpec((NC, RB, FH), lambda i: (0, i, 0)),
            pl.BlockSpec((RB, F), lambda i: (i, 0)),
            pl.BlockSpec((RB, 1), lambda i: (i, 0)),
            pl.BlockSpec((F, F), lambda i: (0, 0)),
            pl.BlockSpec((1, F), lambda i: (0, 0)),
        ],
        out_specs=pl.BlockSpec((RB, F), lambda i: (i, 0)),
        out_shape=jax.ShapeDtypeStruct((N, F), jnp.float32),
    )(s1, u1, dinv, W1, b1)


def _final_body(s_ref, u2_ref, dinv_ref, batch_ref, W2_ref, b2_ref, Wg_ref,
                bg_ref, Wf2_ref, bf2_ref, gl_ref, fl_ref, psum, cnt):
    i = pl.program_id(0)

    @pl.when(i == 0)
    def _():
        psum[...] = jnp.zeros_like(psum)
        cnt[...] = jnp.zeros_like(cnt)

    sp = s_ref[...]
    s_full = jnp.concatenate([sp[0], sp[1]], axis=-1)
    t2 = dinv_ref[...] * (s_full + u2_ref[...])
    gids = lax.broadcasted_iota(jnp.int32, (1, G), 1)
    m = (batch_ref[...] == gids).astype(jnp.float32)
    mt_t2 = lax.dot_general(m, t2, (((0,), (0,)), ((), ())), precision=_P,
                            preferred_element_type=jnp.float32)
    ones_col = jnp.ones((RB, 1), jnp.float32)
    mt_1 = lax.dot_general(m, ones_col, (((0,), (0,)), ((), ())), precision=_P,
                           preferred_element_type=jnp.float32)
    psum[...] += mt_t2
    cnt[...] += mt_1

    @pl.when(i == NB - 1)
    def _():
        pooled128 = psum[...] / jnp.maximum(cnt[...], 1.0)
        pooled = _dot(pooled128, W2_ref[...]) + b2_ref[...]
        gl_ref[...] = _dot(pooled, Wg_ref[...]) + bg_ref[...]
        fl_ref[...] = _dot(pooled, Wf2_ref[...]) + bf2_ref[...]


def _final_call(s2, u2, dinv, batch2d, W2, b2, Wg, bg, Wf2, bf2):
    return pl.pallas_call(
        _final_body,
        grid=(NB,),
        in_specs=[
            pl.BlockSpec((NC, RB, FH), lambda i: (0, i, 0)),
            pl.BlockSpec((RB, F), lambda i: (i, 0)),
            pl.BlockSpec((RB, 1), lambda i: (i, 0)),
            pl.BlockSpec((RB, 1), lambda i: (i, 0)),
            pl.BlockSpec((F, 256), lambda i: (0, 0)),
            pl.BlockSpec((1, 256), lambda i: (0, 0)),
            pl.BlockSpec((256, 8), lambda i: (0, 0)),
            pl.BlockSpec((1, 8), lambda i: (0, 0)),
            pl.BlockSpec((256, G), lambda i: (0, 0)),
            pl.BlockSpec((1, G), lambda i: (0, 0)),
        ],
        out_specs=[
            pl.BlockSpec((G, 8), lambda i: (0, 0)),
            pl.BlockSpec((G, G), lambda i: (0, 0)),
        ],
        out_shape=[
            jax.ShapeDtypeStruct((G, 8), jnp.float32),
            jax.ShapeDtypeStruct((G, G), jnp.float32),
        ],
        scratch_shapes=[
            pltpu.VMEM((G, F), jnp.float32),
            pltpu.VMEM((G, 1), jnp.float32),
        ],
    )(s2, u2, dinv, batch2d, W2, b2, Wg, bg, Wf2, bf2)


def kernel(x, edge_index, batch, W1, b1, W2, b2, Wg, bg, Wf, bf):
    ei = edge_index.astype(jnp.int32)
    row = ei[0]
    col = ei[1]

    hist = _deg_kernel(col)
    u1, dinv = _prep_call(hist, x)
    s1 = _scatter_kernel(u1.reshape(2 * N, FH), row, col)
    u2 = _mid_call(s1, u1, dinv, W1, b1.reshape(1, F))
    s2 = _scatter_kernel(u2.reshape(2 * N, FH), row, col)

    Wf2 = jnp.transpose(Wf, (1, 0, 2)).reshape(256, G)
    gl, fl = _final_call(
        s2, u2, dinv, batch.astype(jnp.int32).reshape(N, 1),
        W2, b2.reshape(1, 256), Wg, bg.reshape(1, 8), Wf2, bf.reshape(1, G))
    return gl, fl.reshape(G, 8, 8)


# speedup vs baseline: 12.2848x; 1.2480x over previous
import functools

import jax
import jax.numpy as jnp
from jax import lax
from jax.experimental import pallas as pl
from jax.experimental.pallas import tpu as pltpu
from jax.experimental.pallas import tpu_sc as plsc

N = 10000
N_PAD = 10240
E = 320000
F = 128
FH = F // 2
FQ = F // 4
G = 64
NC = 2
NS = 16
CH = 128
NCH = 160
E_PAD = NS * NCH * CH
DEPTH = 8
ROWS_PER_TILE = N_PAD // NS
HW = 16
CPW_DEG = NS * NCH // (NC * NS)

_mesh = plsc.VectorSubcoreMesh(core_axis_name="c", subcore_axis_name="s")


@functools.partial(
    pl.kernel,
    mesh=_mesh,
    compiler_params=pltpu.CompilerParams(use_tc_tiling_on_sc=False),
    out_type=jax.ShapeDtypeStruct((NC, N_PAD, HW), jnp.float32),
    scratch_types=(
        [pltpu.VMEM((2 * DEPTH, CH), jnp.int32),
         pltpu.VMEM((CH, HW), jnp.float32),
         pltpu.VMEM((ROWS_PER_TILE, HW), jnp.float32),
         pltpu.VMEM_SHARED((N_PAD, HW), jnp.float32)]
        + [pltpu.SemaphoreType.DMA for _ in range(DEPTH)]
    ),
)
def _deg_kernel(packed_hbm, out_hbm, *scr):
    idxbig, ones_v, stage_v, hist_sh = scr[:4]
    ssems = scr[4:]
    c = lax.axis_index("c")
    s = lax.axis_index("s")

    one16 = jnp.ones((HW,), jnp.float32)
    zero16 = jnp.zeros((HW,), jnp.float32)

    def init_rows(i, _):
        ones_v[i, :] = one16
        return 0

    lax.fori_loop(0, CH, init_rows, 0)

    def zero_rows(i, _):
        stage_v[i, :] = zero16
        return 0

    lax.fori_loop(0, ROWS_PER_TILE, zero_rows, 0)

    my_rows = pl.multiple_of(s * ROWS_PER_TILE, 8)
    pltpu.sync_copy(stage_v, hist_sh.at[pl.ds(my_rows, ROWS_PER_TILE)])
    plsc.subcore_barrier()

    row0 = (c * NS + s) * CPW_DEG

    def group(g, _):
        start = pl.multiple_of(2 * (row0 + g * DEPTH), 8)
        pltpu.sync_copy(packed_hbm.at[pl.ds(start, 2 * DEPTH)], idxbig)
        descs = []
        for b in range(DEPTH):
            descs.append(pltpu.async_copy(
                ones_v, hist_sh.at[idxbig.at[2 * b + 1]], ssems[b], add=True))
        for d in descs:
            d.wait()
        return 0

    lax.fori_loop(0, CPW_DEG // DEPTH, group, 0)
    plsc.subcore_barrier()

    pltpu.sync_copy(hist_sh.at[pl.ds(my_rows, ROWS_PER_TILE)], stage_v)
    pltpu.sync_copy(stage_v, out_hbm.at[c, pl.ds(my_rows, ROWS_PER_TILE)])


@functools.partial(
    pl.kernel,
    mesh=_mesh,
    compiler_params=pltpu.CompilerParams(use_tc_tiling_on_sc=False),
    out_type=jax.ShapeDtypeStruct((NC, 2, N_PAD, FQ), jnp.float32),
    scratch_types=(
        [pltpu.VMEM((2 * NCH, CH), jnp.int32)]
        + [pltpu.VMEM((CH, FQ), jnp.float32) for _ in range(DEPTH)]
        + [pltpu.VMEM((ROWS_PER_TILE, FQ), jnp.float32),
           pltpu.VMEM_SHARED((N_PAD, FQ), jnp.float32)]
        + [pltpu.SemaphoreType.DMA for _ in range(2 * DEPTH)]
    ),
)
def _scatter_kernel(u4_hbm, packed_hbm, out_hbm, *scr):
    idxall = scr[0]
    rowbufs = scr[1:1 + DEPTH]
    stage_v = scr[1 + DEPTH]
    acc_sh = scr[2 + DEPTH]
    gsems = scr[3 + DEPTH:3 + 2 * DEPTH]
    ssems = scr[3 + 2 * DEPTH:]
    c = lax.axis_index("c")
    s = lax.axis_index("s")

    zero16 = jnp.zeros((16,), jnp.float32)
    my_rows = pl.multiple_of(s * ROWS_PER_TILE, 8)
    NG = NCH // DEPTH

    for p in range(2):
        pltpu.sync_copy(packed_hbm.at[pl.ds(pl.multiple_of(2 * s * NCH, 8),
                                            2 * NCH)], idxall)

        def zero_rows(i, _):
            for j in range(FQ // 16):
                stage_v[i, pl.ds(j * 16, 16)] = zero16
            return 0

        lax.fori_loop(0, ROWS_PER_TILE, zero_rows, 0)
        pltpu.sync_copy(stage_v, acc_sh.at[pl.ds(my_rows, ROWS_PER_TILE)])
        plsc.subcore_barrier()

        q = c * 2 + p

        def group(g, _):
            jrows = []
            for b in range(DEPTH):
                j = g * DEPTH + b
                @pl.when(g > 0)
                def _():
                    pltpu.make_async_copy(
                        rowbufs[b], acc_sh.at[idxall.at[0]], ssems[b]).wait()
                for k in range(CH // 16):
                    sl = pl.ds(k * 16, 16)
                    idxall[2 * j, sl] = idxall[2 * j, sl] * 4 + q
                gdescs_b = pltpu.async_copy(
                    u4_hbm.at[idxall.at[2 * j]], rowbufs[b], gsems[b])
                jrows.append((j, gdescs_b))
            for b in range(DEPTH):
                j, gd = jrows[b]
                gd.wait()
                pltpu.async_copy(
                    rowbufs[b], acc_sh.at[idxall.at[2 * j + 1]], ssems[b],
                    add=True)
            return 0

        lax.fori_loop(0, NG, group, 0)
        for b in range(DEPTH):
            pltpu.make_async_copy(
                rowbufs[b], acc_sh.at[idxall.at[0]], ssems[b]).wait()
        plsc.subcore_barrier()

        pltpu.sync_copy(acc_sh.at[pl.ds(my_rows, ROWS_PER_TILE)], stage_v)
        pltpu.sync_copy(stage_v,
                        out_hbm.at[c, p, pl.ds(my_rows, ROWS_PER_TILE)])


RB = 2000
NB = N // RB
_P = jax.lax.Precision.HIGHEST


def _dot(a, b):
    return lax.dot_general(a, b, (((1,), (0,)), ((), ())), precision=_P,
                           preferred_element_type=jnp.float32)


def _prep_body(hist_ref, x_ref, u1_ref, dinv_ref):
    h = hist_ref[...]
    deg = 1.0 + h[0, :, 0:1] + h[1, :, 0:1]
    dinv = lax.rsqrt(deg)
    dinv_ref[...] = dinv
    u1_ref[...] = x_ref[...] * dinv


def _prep_call(hist, x):
    return pl.pallas_call(
        _prep_body,
        grid=(NB,),
        in_specs=[
            pl.BlockSpec((NC, RB, HW), lambda i: (0, i, 0)),
            pl.BlockSpec((RB, F), lambda i: (i, 0)),
        ],
        out_specs=[
            pl.BlockSpec((RB, F), lambda i: (i, 0)),
            pl.BlockSpec((RB, 1), lambda i: (i, 0)),
        ],
        out_shape=[
            jax.ShapeDtypeStruct((N, F), jnp.float32),
            jax.ShapeDtypeStruct((N, 1), jnp.float32),
        ],
    )(hist, x)


def _mid_body(s_ref, u1_ref, dinv_ref, W1_ref, b1_ref, u2_ref):
    sp = s_ref[...]
    s_full = jnp.concatenate([sp[0, 0], sp[0, 1], sp[1, 0], sp[1, 1]],
                             axis=-1)
    dinv = dinv_ref[...]
    t1 = dinv * (s_full + u1_ref[...])
    h = jnp.maximum(_dot(t1, W1_ref[...]) + b1_ref[...], 0.0)
    u2_ref[...] = dinv * h


def _mid_call(s1, u1, dinv, W1, b1):
    return pl.pallas_call(
        _mid_body,
        grid=(NB,),
        in_specs=[
            pl.BlockSpec((NC, 2, RB, FQ), lambda i: (0, 0, i, 0)),
            pl.BlockSpec((RB, F), lambda i: (i, 0)),
            pl.BlockSpec((RB, 1), lambda i: (i, 0)),
            pl.BlockSpec((F, F), lambda i: (0, 0)),
            pl.BlockSpec((1, F), lambda i: (0, 0)),
        ],
        out_specs=pl.BlockSpec((RB, F), lambda i: (i, 0)),
        out_shape=jax.ShapeDtypeStruct((N, F), jnp.float32),
    )(s1, u1, dinv, W1, b1)


def _final_body(s_ref, u2_ref, dinv_ref, batch_ref, W2_ref, b2_ref, Wg_ref,
                bg_ref, Wf2_ref, bf2_ref, gl_ref, fl_ref, psum, cnt):
    i = pl.program_id(0)

    @pl.when(i == 0)
    def _():
        psum[...] = jnp.zeros_like(psum)
        cnt[...] = jnp.zeros_like(cnt)

    sp = s_ref[...]
    s_full = jnp.concatenate([sp[0, 0], sp[0, 1], sp[1, 0], sp[1, 1]],
                             axis=-1)
    t2 = dinv_ref[...] * (s_full + u2_ref[...])
    gids = lax.broadcasted_iota(jnp.int32, (1, G), 1)
    m = (batch_ref[...] == gids).astype(jnp.float32)
    mt_t2 = lax.dot_general(m, t2, (((0,), (0,)), ((), ())), precision=_P,
                            preferred_element_type=jnp.float32)
    ones_col = jnp.ones((RB, 1), jnp.float32)
    mt_1 = lax.dot_general(m, ones_col, (((0,), (0,)), ((), ())), precision=_P,
                           preferred_element_type=jnp.float32)
    psum[...] += mt_t2
    cnt[...] += mt_1

    @pl.when(i == NB - 1)
    def _():
        pooled128 = psum[...] / jnp.maximum(cnt[...], 1.0)
        pooled = _dot(pooled128, W2_ref[...]) + b2_ref[...]
        gl_ref[...] = _dot(pooled, Wg_ref[...]) + bg_ref[...]
        fl_ref[...] = _dot(pooled, Wf2_ref[...]) + bf2_ref[...]


def _final_call(s2, u2, dinv, batch2d, W2, b2, Wg, bg, Wf2, bf2):
    return pl.pallas_call(
        _final_body,
        grid=(NB,),
        in_specs=[
            pl.BlockSpec((NC, 2, RB, FQ), lambda i: (0, 0, i, 0)),
            pl.BlockSpec((RB, F), lambda i: (i, 0)),
            pl.BlockSpec((RB, 1), lambda i: (i, 0)),
            pl.BlockSpec((RB, 1), lambda i: (i, 0)),
            pl.BlockSpec((F, 256), lambda i: (0, 0)),
            pl.BlockSpec((1, 256), lambda i: (0, 0)),
            pl.BlockSpec((256, 8), lambda i: (0, 0)),
            pl.BlockSpec((1, 8), lambda i: (0, 0)),
            pl.BlockSpec((256, G), lambda i: (0, 0)),
            pl.BlockSpec((1, G), lambda i: (0, 0)),
        ],
        out_specs=[
            pl.BlockSpec((G, 8), lambda i: (0, 0)),
            pl.BlockSpec((G, G), lambda i: (0, 0)),
        ],
        out_shape=[
            jax.ShapeDtypeStruct((G, 8), jnp.float32),
            jax.ShapeDtypeStruct((G, G), jnp.float32),
        ],
        scratch_shapes=[
            pltpu.VMEM((G, F), jnp.float32),
            pltpu.VMEM((G, 1), jnp.float32),
        ],
    )(s2, u2, dinv, batch2d, W2, b2, Wg, bg, Wf2, bf2)


def kernel(x, edge_index, batch, W1, b1, W2, b2, Wg, bg, Wf, bf):
    ei = edge_index.astype(jnp.int32)
    row = ei[0]
    col = ei[1]

    npad = E_PAD - E
    row_pad = jnp.concatenate([row, jnp.zeros((npad,), jnp.int32)])
    col_pad = jnp.concatenate([col, jnp.full((npad,), N_PAD - 1, jnp.int32)])
    packed = jnp.stack(
        [row_pad.reshape(NS * NCH, CH), col_pad.reshape(NS * NCH, CH)],
        axis=1).reshape(2 * NS * NCH, CH)

    hist = _deg_kernel(packed)
    u1, dinv = _prep_call(hist, x)
    s1 = _scatter_kernel(u1.reshape(4 * N, FQ), packed)
    u2 = _mid_call(s1, u1, dinv, W1, b1.reshape(1, F))
    s2 = _scatter_kernel(u2.reshape(4 * N, FQ), packed)

    Wf2 = jnp.transpose(Wf, (1, 0, 2)).reshape(256, G)
    gl, fl = _final_call(
        s2, u2, dinv, batch.astype(jnp.int32).reshape(N, 1),
        W2, b2.reshape(1, 256), Wg, bg.reshape(1, 8), Wf2, bf.reshape(1, G))
    return gl, fl.reshape(G, 8, 8)
